# baseline (speedup 1.0000x reference)
import functools

import jax
import jax.numpy as jnp
from jax import lax
from jax.experimental import pallas as pl
from jax.experimental.pallas import tpu as pltpu
from jax.experimental.pallas import tpu_sc as plsc

_D = 128
_PAIRS = _D // 2
_NB = 2
_NCLS = 5
_C = 128
_NS = 16
_SC0_FRAC = 0.50


def _pack_pairs(h):
    hb = lax.bitcast_convert_type(h.astype(jnp.bfloat16), jnp.uint16)
    lo = hb[:, :_PAIRS].astype(jnp.int32)
    hi = hb[:, _PAIRS:].astype(jnp.int32)
    return lo | (hi << 16)


def _mm_body(u_ref, p_ref, o_ref):
    u = u_ref[...]
    h0 = lax.dot_general(u, p_ref[0], (((1,), (0,)), ((), ())),
                         preferred_element_type=jnp.float32)
    h1 = lax.dot_general(u, p_ref[1], (((1,), (0,)), ((), ())),
                         preferred_element_type=jnp.float32)
    o_ref[:, 0:_PAIRS] = _pack_pairs(h0)
    o_ref[:, _PAIRS:2 * _PAIRS] = _pack_pairs(h1)


def _compute_uh(ufeat, P):
    n, d = ufeat.shape
    blk = 1000
    return pl.pallas_call(
        _mm_body,
        grid=(n // blk,),
        in_specs=[
            pl.BlockSpec((blk, d), lambda i: (i, 0)),
            pl.BlockSpec(P.shape, lambda i: (0, 0, 0)),
        ],
        out_specs=pl.BlockSpec((blk, _NB * _PAIRS), lambda i: (i, 0)),
        out_shape=jax.ShapeDtypeStruct((n, _NB * _PAIRS), jnp.int32),
    )(ufeat, P)


def _split(total):
    s0_total = int(round(total * _SC0_FRAC))
    n0, r0 = divmod(s0_total, _NS)
    n1, r1 = divmod(total - s0_total, _NS)
    return n0, r0, n1, r1


def _sc_body(split, uh_hbm, if_hbm, src_hbm, dst_hbm, w_hbm, out_hbm,
             srci, dsti, uhv0, uhv1, ifv0, ifv1, wv, outv0, outv1,
             su0, su1, si0, si1, so0, so1):
    n0, r0, n1, r1 = split
    max0 = n0 + (1 if r0 else 0)
    max1 = n1 + (1 if r1 else 0)
    s0_total = _NS * n0 + r0
    c_idx = lax.axis_index("c")
    s_idx = lax.axis_index("s")
    uhv = [uhv0, uhv1]
    ifv = [ifv0, ifv1]
    outv = [outv0, outv1]
    su = [su0, su1]
    si = [si0, si1]
    so = [so0, so1]

    on_core0 = c_idx == 0
    rows0 = jnp.where(
        on_core0,
        s_idx * n0 + jnp.minimum(s_idx, r0),
        s0_total + s_idx * n1 + jnp.minimum(s_idx, r1))
    m = jnp.where(on_core0,
                  n0 + (s_idx < r0).astype(jnp.int32),
                  n1 + (s_idx < r1).astype(jnp.int32))

    pltpu.sync_copy(w_hbm, wv)

    @pl.when(on_core0)
    def _():
        pltpu.sync_copy(src_hbm.at[pl.ds(rows0, max0)],
                        srci.at[pl.ds(0, max0)])
        pltpu.sync_copy(dst_hbm.at[pl.ds(rows0, max0)],
                        dsti.at[pl.ds(0, max0)])

    @pl.when(jnp.logical_not(on_core0))
    def _():
        pltpu.sync_copy(src_hbm.at[pl.ds(rows0, max1)],
                        srci.at[pl.ds(0, max1)])
        pltpu.sync_copy(dst_hbm.at[pl.ds(rows0, max1)],
                        dsti.at[pl.ds(0, max1)])

    wrows = [wv[i, :] for i in range(_NB)]
    w = [[wrows[i][c] for c in range(_NCLS)] for i in range(_NB)]
    lanes = lax.iota(jnp.int32, 16)
    zero = jnp.zeros((16,), jnp.float32)
    ngr = _C // 16
    evecs = [g * 16 + lanes for g in range(ngr)]
    tile_base = rows0 * _C

    def issue_gathers(k, p):
        pltpu.async_copy(uh_hbm.at[srci.at[k]], uhv[p], su[p])
        pltpu.async_copy(if_hbm.at[dsti.at[k]], ifv[p], si[p])

    def wait_gathers(k, p):
        pltpu.make_async_copy(uh_hbm.at[srci.at[k]], uhv[p], su[p]).wait()
        pltpu.make_async_copy(if_hbm.at[dsti.at[k]], ifv[p], si[p]).wait()

    def out_slice(k):
        base = pl.multiple_of((tile_base + k * _C) * _NCLS, _C)
        return out_hbm.at[pl.ds(base, _C * _NCLS)]

    def wait_out(k, p):
        pltpu.make_async_copy(outv[p], out_slice(k), so[p]).wait()

    def unpack2(x):
        return plsc.unpack(plsc.bitcast(x, jnp.bfloat16),
                           format=plsc.PackFormat.INTERLEAVED,
                           preferred_element_type=jnp.float32)

    def compute(k, p):
        uhr = uhv[p]
        ifr = ifv[p]

        def fstep(f, carry):
            accs = list(carry)
            colv = (lanes + f) & (_PAIRS - 1)
            colb = colv + _PAIRS
            for g in range(ngr):
                u0p = plsc.load_gather(uhr, [evecs[g], colv])
                u1p = plsc.load_gather(uhr, [evecs[g], colb])
                iva = plsc.load_gather(ifr, [evecs[g], colv])
                ivb = plsc.load_gather(ifr, [evecs[g], colb])
                u0a, u0b = unpack2(u0p)
                u1a, u1b = unpack2(u1p)
                accs[2 * g] = accs[2 * g] + u0a * iva + u0b * ivb
                accs[2 * g + 1] = accs[2 * g + 1] + u1a * iva + u1b * ivb
            return tuple(accs)

        res = lax.fori_loop(0, _PAIRS, fstep, (zero,) * (2 * ngr))
        for g in range(ngr):
            a0 = res[2 * g]
            a1 = res[2 * g + 1]
            base5 = evecs[g] * _NCLS
            for c in range(_NCLS):
                ov = w[0][c] * a0 + w[1][c] * a1
                plsc.store_scatter(outv[p], [base5 + c], ov)
        pltpu.async_copy(outv[p], out_slice(k), so[p])

    issue_gathers(0, 0)

    def body(jj, _):
        a = 2 * jj
        b = a + 1
        issue_gathers(b, 1)
        wait_gathers(a, 0)

        @pl.when(jj > 0)
        def _():
            wait_out(a - 2, 0)

        compute(a, 0)

        @pl.when(b + 1 < m)
        def _():
            issue_gathers(b + 1, 0)

        wait_gathers(b, 1)

        @pl.when(jj > 0)
        def _():
            wait_out(b - 2, 1)

        compute(b, 1)
        return 0

    lax.fori_loop(0, m // 2, body, 0)

    is_odd = (m & 1) == 1

    @pl.when(is_odd)
    def _():
        wait_gathers(m - 1, 0)
        wait_out(m - 3, 0)
        compute(m - 1, 0)
        wait_out(m - 2, 1)
        wait_out(m - 1, 0)

    @pl.when(jnp.logical_not(is_odd))
    def _():
        wait_out(m - 2, 0)
        wait_out(m - 1, 1)


def _sc_scores(uh, ifeat, src2d, dst2d, w2, e_pad, split):
    n0, r0 = split[0], split[1]
    max0 = n0 + (1 if r0 else 0)
    mesh = plsc.VectorSubcoreMesh(core_axis_name="c", subcore_axis_name="s")
    f = pl.kernel(
        functools.partial(_sc_body, split),
        mesh=mesh,
        compiler_params=pltpu.CompilerParams(
            needs_layout_passes=False, use_tc_tiling_on_sc=False),
        out_type=jax.ShapeDtypeStruct((e_pad * _NCLS,), jnp.float32),
        scratch_types=[
            pltpu.VMEM((max0, _C), jnp.int32),
            pltpu.VMEM((max0, _C), jnp.int32),
            pltpu.VMEM((_C, _NB * _PAIRS), jnp.int32),
            pltpu.VMEM((_C, _NB * _PAIRS), jnp.int32),
            pltpu.VMEM((_C, _D), jnp.float32),
            pltpu.VMEM((_C, _D), jnp.float32),
            pltpu.VMEM((_NB, 16), jnp.float32),
            pltpu.VMEM((_C * _NCLS,), jnp.float32),
            pltpu.VMEM((_C * _NCLS,), jnp.float32),
            pltpu.SemaphoreType.DMA,
            pltpu.SemaphoreType.DMA,
            pltpu.SemaphoreType.DMA,
            pltpu.SemaphoreType.DMA,
            pltpu.SemaphoreType.DMA,
            pltpu.SemaphoreType.DMA,
        ],
    )
    return f(uh, ifeat, src2d, dst2d, w2)


def kernel(ufeat, ifeat, edge_index, P, W_combine):
    e = edge_index.shape[1]
    uh_i32 = _compute_uh(ufeat, P)
    src = edge_index[0].astype(jnp.int32)
    dst = edge_index[1].astype(jnp.int32)
    e_pad = -(-e // _C) * _C
    if e_pad != e:
        src = jnp.pad(src, (0, e_pad - e))
        dst = jnp.pad(dst, (0, e_pad - e))
    src2d = src.reshape(e_pad // _C, _C)
    dst2d = dst.reshape(e_pad // _C, _C)
    split = _split(e_pad // _C)
    w2 = jnp.zeros((_NB, 16), jnp.float32).at[:, :_NCLS].set(W_combine.T)
    out = _sc_scores(uh_i32, ifeat, src2d, dst2d, w2, e_pad, split)
    return out.reshape(e_pad, _NCLS)[:e]


# speedup vs baseline: 1.1359x; 1.1359x over previous
import functools

import jax
import jax.numpy as jnp
from jax import lax
from jax.experimental import pallas as pl
from jax.experimental.pallas import tpu as pltpu
from jax.experimental.pallas import tpu_sc as plsc

_D = 128
_PAIRS = _D // 2
_NB = 2
_NCLS = 5
_C = 128
_NS = 16
_SC0_FRAC = 0.50


def _pack_pairs(h):
    hb = lax.bitcast_convert_type(h.astype(jnp.bfloat16), jnp.uint16)
    lo = hb[:, :_PAIRS].astype(jnp.int32)
    hi = hb[:, _PAIRS:].astype(jnp.int32)
    return lo | (hi << 16)


def _mm_body(u_ref, p_ref, o_ref):
    u = u_ref[...]
    h0 = lax.dot_general(u, p_ref[0], (((1,), (0,)), ((), ())),
                         preferred_element_type=jnp.float32)
    h1 = lax.dot_general(u, p_ref[1], (((1,), (0,)), ((), ())),
                         preferred_element_type=jnp.float32)
    o_ref[:, 0:_PAIRS] = _pack_pairs(h0)
    o_ref[:, _PAIRS:2 * _PAIRS] = _pack_pairs(h1)


def _compute_uh(ufeat, P):
    n, d = ufeat.shape
    blk = 1000
    return pl.pallas_call(
        _mm_body,
        grid=(n // blk,),
        in_specs=[
            pl.BlockSpec((blk, d), lambda i: (i, 0)),
            pl.BlockSpec(P.shape, lambda i: (0, 0, 0)),
        ],
        out_specs=pl.BlockSpec((blk, _NB * _PAIRS), lambda i: (i, 0)),
        out_shape=jax.ShapeDtypeStruct((n, _NB * _PAIRS), jnp.int32),
    )(ufeat, P)


def _split(total):
    s0_total = int(round(total * _SC0_FRAC))
    n0, r0 = divmod(s0_total, _NS)
    n1, r1 = divmod(total - s0_total, _NS)
    return n0, r0, n1, r1


def _sc_body(split, uh_hbm, if_hbm, src_hbm, dst_hbm, w_hbm, out_hbm,
             srci, dsti, uhv0, uhv1, ifv0, ifv1, wv, outv0, outv1,
             su0, su1, si0, si1, so0, so1):
    n0, r0, n1, r1 = split
    max0 = n0 + (1 if r0 else 0)
    max1 = n1 + (1 if r1 else 0)
    s0_total = _NS * n0 + r0
    c_idx = lax.axis_index("c")
    s_idx = lax.axis_index("s")
    uhv = [uhv0, uhv1]
    ifv = [ifv0, ifv1]
    outv = [outv0, outv1]
    su = [su0, su1]
    si = [si0, si1]
    so = [so0, so1]

    on_core0 = c_idx == 0
    rows0 = jnp.where(
        on_core0,
        s_idx * n0 + jnp.minimum(s_idx, r0),
        s0_total + s_idx * n1 + jnp.minimum(s_idx, r1))
    m = jnp.where(on_core0,
                  n0 + (s_idx < r0).astype(jnp.int32),
                  n1 + (s_idx < r1).astype(jnp.int32))

    pltpu.sync_copy(w_hbm, wv)

    @pl.when(on_core0)
    def _():
        pltpu.sync_copy(src_hbm.at[pl.ds(rows0, max0)],
                        srci.at[pl.ds(0, max0)])
        pltpu.sync_copy(dst_hbm.at[pl.ds(rows0, max0)],
                        dsti.at[pl.ds(0, max0)])

    @pl.when(jnp.logical_not(on_core0))
    def _():
        pltpu.sync_copy(src_hbm.at[pl.ds(rows0, max1)],
                        srci.at[pl.ds(0, max1)])
        pltpu.sync_copy(dst_hbm.at[pl.ds(rows0, max1)],
                        dsti.at[pl.ds(0, max1)])

    wrows = [wv[i, :] for i in range(_NB)]
    w = [[wrows[i][c] for c in range(_NCLS)] for i in range(_NB)]
    lanes = lax.iota(jnp.int32, 16)
    zero = jnp.zeros((16,), jnp.float32)
    ngr = _C // 16
    evecs = [g * 16 + lanes for g in range(ngr)]
    tile_base = rows0 * _C

    def issue_gathers(k, p):
        pltpu.async_copy(uh_hbm.at[srci.at[k]], uhv[p], su[p])
        pltpu.async_copy(if_hbm.at[dsti.at[k]], ifv[p], si[p])

    def wait_gathers(k, p):
        pltpu.make_async_copy(uh_hbm.at[srci.at[k]], uhv[p], su[p]).wait()
        pltpu.make_async_copy(if_hbm.at[dsti.at[k]], ifv[p], si[p]).wait()

    def out_slice(k):
        base = pl.multiple_of(tile_base + k * _C, _C)
        return out_hbm.at[pl.ds(base, _C), :]

    def wait_out(k, p):
        pltpu.make_async_copy(outv[p], out_slice(k), so[p]).wait()

    def unpack2(x):
        return plsc.unpack(plsc.bitcast(x, jnp.bfloat16),
                           format=plsc.PackFormat.INTERLEAVED,
                           preferred_element_type=jnp.float32)

    def compute(k, p):
        uhr = uhv[p]
        ifr = ifv[p]

        def fstep(f, carry):
            accs = list(carry)
            colv = (lanes + f) & (_PAIRS - 1)
            colb = colv + _PAIRS
            for g in range(ngr):
                u0p = plsc.load_gather(uhr, [evecs[g], colv])
                u1p = plsc.load_gather(uhr, [evecs[g], colb])
                iva = plsc.load_gather(ifr, [evecs[g], colv])
                ivb = plsc.load_gather(ifr, [evecs[g], colb])
                u0a, u0b = unpack2(u0p)
                u1a, u1b = unpack2(u1p)
                accs[2 * g] = accs[2 * g] + u0a * iva + u0b * ivb
                accs[2 * g + 1] = accs[2 * g + 1] + u1a * iva + u1b * ivb
            return tuple(accs)

        res = lax.fori_loop(0, _PAIRS, fstep, (zero,) * (2 * ngr))
        for g in range(ngr):
            a0 = res[2 * g]
            a1 = res[2 * g + 1]
            for c in range(_NCLS):
                ov = w[0][c] * a0 + w[1][c] * a1
                plsc.store_scatter(
                    outv[p], [evecs[g], jnp.full((16,), c, jnp.int32)], ov)
        pltpu.async_copy(outv[p], out_slice(k), so[p])

    issue_gathers(0, 0)

    def body(jj, _):
        a = 2 * jj
        b = a + 1
        issue_gathers(b, 1)
        wait_gathers(a, 0)

        @pl.when(jj > 0)
        def _():
            wait_out(a - 2, 0)

        compute(a, 0)

        @pl.when(b + 1 < m)
        def _():
            issue_gathers(b + 1, 0)

        wait_gathers(b, 1)

        @pl.when(jj > 0)
        def _():
            wait_out(b - 2, 1)

        compute(b, 1)
        return 0

    lax.fori_loop(0, m // 2, body, 0)

    is_odd = (m & 1) == 1

    @pl.when(is_odd)
    def _():
        wait_gathers(m - 1, 0)
        wait_out(m - 3, 0)
        compute(m - 1, 0)
        wait_out(m - 2, 1)
        wait_out(m - 1, 0)

    @pl.when(jnp.logical_not(is_odd))
    def _():
        wait_out(m - 2, 0)
        wait_out(m - 1, 1)


def _sc_scores(uh, ifeat, src2d, dst2d, w2, e_pad, split):
    n0, r0 = split[0], split[1]
    max0 = n0 + (1 if r0 else 0)
    mesh = plsc.VectorSubcoreMesh(core_axis_name="c", subcore_axis_name="s")
    f = pl.kernel(
        functools.partial(_sc_body, split),
        mesh=mesh,
        compiler_params=pltpu.CompilerParams(
            needs_layout_passes=False, use_tc_tiling_on_sc=False),
        out_type=jax.ShapeDtypeStruct((e_pad, _NCLS), jnp.float32),
        scratch_types=[
            pltpu.VMEM((max0, _C), jnp.int32),
            pltpu.VMEM((max0, _C), jnp.int32),
            pltpu.VMEM((_C, _NB * _PAIRS), jnp.int32),
            pltpu.VMEM((_C, _NB * _PAIRS), jnp.int32),
            pltpu.VMEM((_C, _D), jnp.float32),
            pltpu.VMEM((_C, _D), jnp.float32),
            pltpu.VMEM((_NB, 16), jnp.float32),
            pltpu.VMEM((_C, _NCLS), jnp.float32),
            pltpu.VMEM((_C, _NCLS), jnp.float32),
            pltpu.SemaphoreType.DMA,
            pltpu.SemaphoreType.DMA,
            pltpu.SemaphoreType.DMA,
            pltpu.SemaphoreType.DMA,
            pltpu.SemaphoreType.DMA,
            pltpu.SemaphoreType.DMA,
        ],
    )
    return f(uh, ifeat, src2d, dst2d, w2)


def kernel(ufeat, ifeat, edge_index, P, W_combine):
    e = edge_index.shape[1]
    uh_i32 = _compute_uh(ufeat, P)
    src = edge_index[0].astype(jnp.int32)
    dst = edge_index[1].astype(jnp.int32)
    e_pad = -(-e // _C) * _C
    if e_pad != e:
        src = jnp.pad(src, (0, e_pad - e))
        dst = jnp.pad(dst, (0, e_pad - e))
    src2d = src.reshape(e_pad // _C, _C)
    dst2d = dst.reshape(e_pad // _C, _C)
    split = _split(e_pad // _C)
    w2 = jnp.zeros((_NB, 16), jnp.float32).at[:, :_NCLS].set(W_combine.T)
    out = _sc_scores(uh_i32, ifeat, src2d, dst2d, w2, e_pad, split)
    return out[:e]
